# Initial kernel scaffold; baseline (speedup 1.0000x reference)
import jax
import jax.numpy as jnp
from jax.experimental import pallas as pl


def kernel(user_indices, pos_item_indices, neg_item_indices, adj_rows, adj_cols, adj_vals, int_rows, int_cols, int_vals, E0_weight, image_data, text_data, attr_data, prof_data, img_W, img_b, img_gamma, img_beta, txt_W, txt_b, txt_gamma, txt_beta, attr_W, attr_b, attr_gamma, attr_beta, prof_W, prof_b, prof_gamma, prof_beta):
    raise NotImplementedError("write your pallas kernel here")


# speedup vs baseline: 2.9451x; 2.9451x over previous
import dataclasses
import functools

import jax
import jax.numpy as jnp
from jax import lax
from jax.experimental import pallas as pl
from jax.experimental.pallas import tpu as pltpu
from jax.experimental.pallas import tpu_sc as plsc

NU = 25000
NI = 25000
EMB = 64
NS = 16
NC = 2
LANES = 16
KW = 128
NPAD = 25088
RPT = NPAD // NS
BATCH = 4096
BK = BATCH // (NC * NS)

MODEL_CAT_RATE = 0.02
USER_CAT_RATE = 2.8
ITEM_CAT_RATE = 0.005

_f32 = jnp.float32
_i32 = jnp.int32


def _sc_params():
    cp = pltpu.CompilerParams()
    for field, val in (("needs_layout_passes", False),
                       ("use_tc_tiling_on_sc", False)):
        if field in pltpu.CompilerParams.__dataclass_fields__:
            cp = dataclasses.replace(cp, **{field: val})
    return cp


def _spmm_pair(rows2d, cols2d, vals2d, table0, o0, table1, o1):
    nwin_tot = rows2d.shape[0]
    wpt = nwin_tot // NS
    ch = 8
    mesh = plsc.VectorSubcoreMesh(core_axis_name="c", subcore_axis_name="s")
    nz = RPT - (RPT // KW) * KW

    @functools.partial(
        pl.kernel,
        mesh=mesh,
        out_type=[jax.ShapeDtypeStruct((NPAD, EMB), _f32),
                  jax.ShapeDtypeStruct((NPAD, EMB), _f32)],
        scratch_types=[
            pltpu.VMEM_SHARED((NPAD, EMB), _f32),
            pltpu.VMEM((ch, KW), _i32),
            pltpu.VMEM((ch, KW), _i32),
            pltpu.VMEM((ch, KW), _f32),
            pltpu.VMEM((KW, EMB), _f32),
        ],
        compiler_params=_sc_params(),
    )
    def k(rows_hbm, cols_hbm, vals_hbm, t0_hbm, t1_hbm, out0, out1,
          acc, dstv, srcv, valv, gbuf):
        c = lax.axis_index("c")
        s = lax.axis_index("s")

        def run(table, out, orient):
            dsth, srch = ((rows_hbm, cols_hbm) if orient == "A"
                          else (cols_hbm, rows_hbm))

            @pl.loop(0, KW)
            def _(r):
                for j in range(EMB // LANES):
                    gbuf[r, pl.ds(j * LANES, LANES)] = jnp.zeros((LANES,), _f32)

            @pl.loop(0, RPT // KW)
            def _(z):
                pltpu.sync_copy(gbuf, acc.at[pl.ds(s * RPT + z * KW, KW)])
            if nz:
                pltpu.sync_copy(gbuf.at[pl.ds(0, nz)],
                                acc.at[pl.ds(s * RPT + (RPT // KW) * KW, nz)])
            plsc.subcore_barrier()

            @pl.loop(0, wpt, step=ch)
            def _(w0):
                base = s * wpt + w0
                pltpu.sync_copy(dsth.at[pl.ds(base, ch)], dstv)
                pltpu.sync_copy(srch.at[pl.ds(base, ch)], srcv)
                pltpu.sync_copy(vals_hbm.at[pl.ds(base, ch)], valv)
                for wi in range(ch):
                    pltpu.sync_copy(table.at[srcv.at[wi]], gbuf)
                    wb = jnp.zeros((LANES,), _i32) + wi

                    @pl.loop(0, KW)
                    def _(e):
                        eb = jnp.zeros((LANES,), _i32) + e
                        vb = plsc.load_gather(valv, [wb, eb])
                        for j in range(EMB // LANES):
                            sl = (e, pl.ds(j * LANES, LANES))
                            gbuf[sl] = gbuf[sl] * vb

                    pltpu.sync_copy(gbuf, acc.at[dstv.at[wi]], add=True)

            plsc.subcore_barrier()
            pltpu.sync_copy(acc.at[pl.ds(s * RPT, RPT)],
                            out.at[pl.ds(s * RPT, RPT)])

        @pl.when(c == 0)
        def _():
            run(t0_hbm, out0, o0)

        @pl.when(c == 1)
        def _():
            run(t1_hbm, out1, o1)

    return k(rows2d, cols2d, vals2d, table0, table1)


def _gather21(tables, pairs, idx_u, idx_p, idx_n):
    mesh = plsc.VectorSubcoreMesh(core_axis_name="c", subcore_axis_name="s")
    nt = len(tables)

    @functools.partial(
        pl.kernel,
        mesh=mesh,
        out_type=[jax.ShapeDtypeStruct((BATCH, EMB), _f32)] * len(pairs),
        scratch_types=[pltpu.VMEM((BK,), _i32)] * 3
        + [pltpu.VMEM((BK, EMB), _f32)],
        compiler_params=_sc_params(),
    )
    def k(*refs):
        tabs = refs[:nt]
        idxs = refs[nt:nt + 3]
        outs = refs[nt + 3:nt + 3 + len(pairs)]
        iv = refs[nt + 3 + len(pairs):nt + 6 + len(pairs)]
        gbuf = refs[-1]
        c = lax.axis_index("c")
        s = lax.axis_index("s")
        base = (c * NS + s) * BK
        for j in range(3):
            pltpu.sync_copy(idxs[j].at[pl.ds(base, BK)], iv[j])
        for o, (ti, ii) in zip(outs, pairs):
            pltpu.sync_copy(tabs[ti].at[iv[ii]], gbuf)
            pltpu.sync_copy(gbuf, o.at[pl.ds(base, BK)])

    return k(*tables, idx_u, idx_p, idx_n)


def _mm_bn_stats(x, w, b, gamma, beta):
    nr, d = x.shape
    br = 1000
    nb = nr // br

    def body(x_ref, w_ref, b_ref, g_ref, be_ref, y_ref, st_ref, acc_ref):
        i = pl.program_id(0)
        y = jnp.dot(x_ref[...], w_ref[...],
                    preferred_element_type=_f32) + b_ref[...]
        y_ref[...] = y

        @pl.when(i == 0)
        def _():
            acc_ref[...] = jnp.zeros_like(acc_ref)

        acc_ref[0:1, :] += jnp.sum(y, axis=0, keepdims=True)
        acc_ref[1:2, :] += jnp.sum(y * y, axis=0, keepdims=True)

        @pl.when(i == nb - 1)
        def _():
            mu = acc_ref[0:1, :] * (1.0 / nr)
            var = acc_ref[1:2, :] * (1.0 / nr) - mu * mu
            a = g_ref[...] * lax.rsqrt(var + 1e-5)
            st_ref[0:1, :] = a
            st_ref[1:2, :] = be_ref[...] - mu * a

    return pl.pallas_call(
        body,
        grid=(nb,),
        in_specs=[
            pl.BlockSpec((br, d), lambda i: (i, 0)),
            pl.BlockSpec((d, EMB), lambda i: (0, 0)),
            pl.BlockSpec((1, EMB), lambda i: (0, 0)),
            pl.BlockSpec((1, EMB), lambda i: (0, 0)),
            pl.BlockSpec((1, EMB), lambda i: (0, 0)),
        ],
        out_specs=[
            pl.BlockSpec((br, EMB), lambda i: (i, 0)),
            pl.BlockSpec((2, EMB), lambda i: (0, 0)),
        ],
        out_shape=[
            jax.ShapeDtypeStruct((nr, EMB), _f32),
            jax.ShapeDtypeStruct((2, EMB), _f32),
        ],
        scratch_shapes=[pltpu.VMEM((2, EMB), _f32)],
    )(x, w, b.reshape(1, EMB), gamma.reshape(1, EMB), beta.reshape(1, EMB))


def _bn_apply4(ys, sts):
    nr = ys[0].shape[0]
    br = 1000
    nb = nr // br

    def body(y0, s0, y1, s1, y2, s2, y3, s3, o0, o1, o2, o3):
        for y, st, o in ((y0, s0, o0), (y1, s1, o1), (y2, s2, o2), (y3, s3, o3)):
            o[...] = y[...] * st[0:1, :] + st[1:2, :]

    in_specs = []
    args = []
    for y, st in zip(ys, sts):
        in_specs += [pl.BlockSpec((br, EMB), lambda i: (i, 0)),
                     pl.BlockSpec((2, EMB), lambda i: (0, 0))]
        args += [y, st]
    return pl.pallas_call(
        body,
        grid=(nb,),
        in_specs=in_specs,
        out_specs=[pl.BlockSpec((br, EMB), lambda i: (i, 0))] * 4,
        out_shape=[jax.ShapeDtypeStruct((nr, EMB), _f32)] * 4,
    )(*args)


def _combine(g):
    br = 512
    nb = BATCH // br

    def body(*refs):
        (e0u, e1u, e2u, uimg, utxt, uprof2, uattr,
         e0p, e1p, e2p, iimg2p, itxt2p, iprofp, iattr2p,
         e0n, e1n, e2n, iimg2n, itxt2n, iprofn, iattr2n, out) = refs

        def l2n(ref):
            x = ref[...]
            n = jnp.sqrt(jnp.sum(x * x, axis=1, keepdims=True))
            return x / jnp.maximum(n, 1e-12)

        third = 1.0 / 3.0
        ue = ((e0u[...] + e1u[...] + e2u[...]) * third
              + MODEL_CAT_RATE * l2n(uimg) + MODEL_CAT_RATE * l2n(utxt)
              + USER_CAT_RATE * l2n(uprof2) + ITEM_CAT_RATE * l2n(uattr))
        ip = ((e0p[...] + e1p[...] + e2p[...]) * third
              + MODEL_CAT_RATE * l2n(iimg2p) + MODEL_CAT_RATE * l2n(itxt2p)
              + USER_CAT_RATE * l2n(iprofp) + ITEM_CAT_RATE * l2n(iattr2p))
        inn = ((e0n[...] + e1n[...] + e2n[...]) * third
               + MODEL_CAT_RATE * l2n(iimg2n) + MODEL_CAT_RATE * l2n(itxt2n)
               + USER_CAT_RATE * l2n(iprofn) + ITEM_CAT_RATE * l2n(iattr2n))
        out[...] = jnp.concatenate([
            ue, ip, inn,
            uimg[...], iimg2p[...], iimg2n[...],
            utxt[...], itxt2p[...], itxt2n[...],
            uprof2[...], iprofp[...], iprofn[...],
        ], axis=1)

    return pl.pallas_call(
        body,
        grid=(nb,),
        in_specs=[pl.BlockSpec((br, EMB), lambda i: (i, 0))] * 21,
        out_specs=pl.BlockSpec((br, 12 * EMB), lambda i: (i, 0)),
        out_shape=jax.ShapeDtypeStruct((BATCH, 12 * EMB), _f32),
    )(*g)


def kernel(user_indices, pos_item_indices, neg_item_indices,
           adj_rows, adj_cols, adj_vals,
           int_rows, int_cols, int_vals,
           E0_weight, image_data, text_data, attr_data, prof_data,
           img_W, img_b, img_gamma, img_beta,
           txt_W, txt_b, txt_gamma, txt_beta,
           attr_W, attr_b, attr_gamma, attr_beta,
           prof_W, prof_b, prof_gamma, prof_beta):
    e = int_rows.shape[0]
    wpt = -(-e // (NS * KW))
    wpt = -(-wpt // 8) * 8
    epad = NS * KW * wpt

    def pad2d(a, dtype):
        a = a.astype(dtype)
        a = jnp.pad(a, (0, epad - e))
        return a.reshape(NS * wpt, KW)

    rows2d = pad2d(int_rows, _i32)
    cols2d = pad2d(int_cols, _i32)
    vals2d = pad2d(int_vals, _f32)
    idx_u = user_indices.astype(_i32)
    idx_p = pos_item_indices.astype(_i32)
    idx_n = neg_item_indices.astype(_i32)

    e0u = E0_weight[:NU]
    e0i = E0_weight[NU:]

    img_y, img_st = _mm_bn_stats(image_data, img_W, img_b, img_gamma, img_beta)
    txt_y, txt_st = _mm_bn_stats(text_data, txt_W, txt_b, txt_gamma, txt_beta)
    attr_y, attr_st = _mm_bn_stats(attr_data, attr_W, attr_b, attr_gamma, attr_beta)
    prof_y, prof_st = _mm_bn_stats(prof_data, prof_W, prof_b, prof_gamma, prof_beta)
    item_img, item_txt, item_attr, user_prof = _bn_apply4(
        (img_y, txt_y, attr_y, prof_y), (img_st, txt_st, attr_st, prof_st))

    e1u, e1i = _spmm_pair(rows2d, cols2d, vals2d, e0i, "A", e0u, "T")
    e2u, e2i = _spmm_pair(rows2d, cols2d, vals2d, e1i, "A", e1u, "T")
    uimg, utxt = _spmm_pair(rows2d, cols2d, vals2d, item_img, "A", item_txt, "A")
    iimg2, itxt2 = _spmm_pair(rows2d, cols2d, vals2d, uimg, "T", utxt, "T")
    uattr, iprof = _spmm_pair(rows2d, cols2d, vals2d, item_attr, "A", user_prof, "T")
    iattr2, uprof2 = _spmm_pair(rows2d, cols2d, vals2d, uattr, "T", iprof, "A")

    tables = (e0u, e1u, e2u, uimg, utxt, uprof2, uattr,
              e0i, e1i, e2i, iimg2, itxt2, iprof, iattr2)
    pairs = ([(0, 0), (1, 0), (2, 0), (3, 0), (4, 0), (5, 0), (6, 0)]
             + [(7, 1), (8, 1), (9, 1), (10, 1), (11, 1), (12, 1), (13, 1)]
             + [(7, 2), (8, 2), (9, 2), (10, 2), (11, 2), (12, 2), (13, 2)])
    g = _gather21(tables, pairs, idx_u, idx_p, idx_n)

    return _combine(g)
